# baseline (speedup 1.0000x reference)
import functools

import jax
import jax.numpy as jnp
from jax import lax
from jax.experimental import pallas as pl
from jax.experimental.pallas import tpu as pltpu
from jax.experimental.pallas import tpu_sc as plsc

D = 64
CHUNK = 128
L = 16
EPS = 1e-5


def _rsqrt(a):
    i = plsc.bitcast(a, jnp.int32)
    i = jnp.int32(0x5F3759DF) - lax.shift_right_logical(i, 1)
    y = plsc.bitcast(i, jnp.float32)
    half = a * 0.5
    for _ in range(3):
        y = y * (1.5 - half * y * y)
    return y


@functools.lru_cache(maxsize=None)
def _make_kernel(n_rows):
    info = plsc.get_sparse_core_info()
    nc, ns = info.num_cores, info.num_subcores
    nw = nc * ns
    rows_per_w = n_rows // nw
    n_chunks = rows_per_w // CHUNK
    assert rows_per_w % CHUNK == 0 and n_rows % nw == 0
    mesh = plsc.VectorSubcoreMesh(core_axis_name="c", subcore_axis_name="s")

    @functools.partial(
        pl.kernel,
        mesh=mesh,
        out_type=jax.ShapeDtypeStruct((n_rows, D), jnp.float32),
        compiler_params=pltpu.CompilerParams(
            use_tc_tiling_on_sc=False, needs_layout_passes=False
        ),
        scratch_types=[
            pltpu.VMEM((CHUNK,), jnp.int32),
            pltpu.VMEM((CHUNK, D), jnp.float32),
            pltpu.VMEM((CHUNK, D), jnp.float32),
            pltpu.VMEM((D,), jnp.float32),
            pltpu.VMEM((D,), jnp.float32),
            pltpu.SemaphoreType.DMA,
        ],
    )
    def k(x_hbm, table_hbm, gamma_hbm, beta_hbm, out_hbm,
          idx_v, rows_v, obuf_v, gamma_v, beta_v, gsem):
        wid = lax.axis_index("s") * nc + lax.axis_index("c")
        base0 = wid * rows_per_w
        pltpu.sync_copy(gamma_hbm, gamma_v)
        pltpu.sync_copy(beta_hbm, beta_v)
        lanes = lax.iota(jnp.int32, L)

        def chunk_body(g, carry):
            base = base0 + g * CHUNK
            pltpu.sync_copy(x_hbm.at[pl.ds(base, CHUNK)], idx_v)
            pltpu.async_copy(table_hbm.at[idx_v], rows_v, gsem).wait()

            def block_body(b, carry2):
                row_idx = b * L + lanes
                s = jnp.zeros((L,), jnp.float32)
                q = jnp.zeros((L,), jnp.float32)
                for j in range(D):
                    cj = jnp.full((L,), j, jnp.int32)
                    v = plsc.load_gather(rows_v, [row_idx, cj])
                    s = s + v
                    q = q + v * v
                mean = s * (1.0 / D)
                var = q * (1.0 / D) - mean * mean
                rstd = _rsqrt(var + EPS)
                mrs = mean * rstd
                for jj in range(D // L):
                    gvec = gamma_v[pl.ds(jj * L, L)]
                    bvec = beta_v[pl.ds(jj * L, L)]
                    for t in range(L):
                        j = jj * L + t
                        cj = jnp.full((L,), j, jnp.int32)
                        v = plsc.load_gather(rows_v, [row_idx, cj])
                        o = (v * rstd - mrs) * gvec[t] + bvec[t]
                        plsc.store_scatter(obuf_v, [row_idx, cj], o)
                return carry2

            lax.fori_loop(0, CHUNK // L, block_body, 0)
            pltpu.sync_copy(obuf_v, out_hbm.at[pl.ds(base, CHUNK)])
            return carry

        lax.fori_loop(0, n_chunks, chunk_body, 0)

    return k


def kernel(x, table, gamma, beta):
    b, s = x.shape
    n = b * s
    out = _make_kernel(n)(x.reshape(n), table, gamma, beta)
    return out.reshape(b, s, D)


# speedup vs baseline: 1.0851x; 1.0851x over previous
import functools

import jax
import jax.numpy as jnp
from jax import lax
from jax.experimental import pallas as pl
from jax.experimental.pallas import tpu as pltpu
from jax.experimental.pallas import tpu_sc as plsc

D = 64
PAD = 64
CHUNK = 128
L = 16
EPS = 1e-5


def _rsqrt(a):
    i = plsc.bitcast(a, jnp.int32)
    i = jnp.int32(0x5F3759DF) - lax.shift_right_logical(i, 1)
    y = plsc.bitcast(i, jnp.float32)
    half = a * 0.5
    for _ in range(3):
        y = y * (1.5 - half * y * y)
    return y


@functools.lru_cache(maxsize=None)
def _make_kernel(n_rows):
    info = plsc.get_sparse_core_info()
    nc, ns = info.num_cores, info.num_subcores
    nw = nc * ns
    rows_per_w = n_rows // nw
    n_chunks = rows_per_w // CHUNK
    n2 = n_chunks // 2
    assert rows_per_w % CHUNK == 0 and n_rows % nw == 0 and n_chunks % 2 == 0
    mesh = plsc.VectorSubcoreMesh(core_axis_name="c", subcore_axis_name="s")

    @functools.partial(
        pl.kernel,
        mesh=mesh,
        out_type=jax.ShapeDtypeStruct((n_rows, D), jnp.float32),
        compiler_params=pltpu.CompilerParams(
            use_tc_tiling_on_sc=False, needs_layout_passes=False
        ),
        scratch_types=[
            pltpu.VMEM((n_chunks, CHUNK), jnp.int32),
            pltpu.VMEM((CHUNK, PAD), jnp.float32),
            pltpu.VMEM((CHUNK, PAD), jnp.float32),
            pltpu.VMEM((CHUNK, PAD), jnp.float32),
            pltpu.VMEM((CHUNK, PAD), jnp.float32),
            pltpu.VMEM((D,), jnp.float32),
            pltpu.VMEM((D,), jnp.float32),
            pltpu.SemaphoreType.DMA,
            pltpu.SemaphoreType.DMA,
            pltpu.SemaphoreType.DMA,
            pltpu.SemaphoreType.DMA,
        ],
    )
    def k(x_hbm, table_hbm, gamma_hbm, beta_hbm, out_hbm,
          idx_v, rows_a, rows_b, obuf_a, obuf_b, gamma_v, beta_v,
          gsem_a, gsem_b, wsem_a, wsem_b):
        wid = lax.axis_index("s") * nc + lax.axis_index("c")
        base0 = wid * rows_per_w
        pltpu.sync_copy(gamma_hbm, gamma_v)
        pltpu.sync_copy(beta_hbm, beta_v)
        pltpu.sync_copy(
            x_hbm.at[pl.ds(wid * n_chunks, n_chunks), :], idx_v
        )
        lanes = lax.iota(jnp.int32, L)

        def gather(g, rows, sem):
            return pltpu.make_async_copy(
                table_hbm.at[idx_v.at[g]], rows.at[:, pl.ds(0, D)], sem
            )

        def compute(rows, obuf):
            def block_body(b, carry2):
                row_idx = b * L + lanes
                s = jnp.zeros((L,), jnp.float32)
                q = jnp.zeros((L,), jnp.float32)
                for j in range(D):
                    cj = jnp.full((L,), j, jnp.int32)
                    v = plsc.load_gather(rows, [row_idx, cj])
                    s = s + v
                    q = q + v * v
                mean = s * (1.0 / D)
                var = q * (1.0 / D) - mean * mean
                rstd = _rsqrt(var + EPS)
                mrs = mean * rstd
                for jj in range(D // L):
                    gvec = gamma_v[pl.ds(jj * L, L)]
                    bvec = beta_v[pl.ds(jj * L, L)]
                    for t in range(L):
                        j = jj * L + t
                        cj = jnp.full((L,), j, jnp.int32)
                        v = plsc.load_gather(rows, [row_idx, cj])
                        o = (v * rstd - mrs) * gvec[t] + bvec[t]
                        plsc.store_scatter(obuf, [row_idx, cj], o)
                return carry2

            lax.fori_loop(0, CHUNK // L, block_body, 0)

        def writeback(g, obuf, sem):
            return pltpu.make_async_copy(
                obuf.at[:, pl.ds(0, D)],
                out_hbm.at[pl.ds(base0 + g * CHUNK, CHUNK)],
                sem,
            )

        gather(0, rows_a, gsem_a).start()

        def body(g2, carry):
            ga = 2 * g2
            gather(ga + 1, rows_b, gsem_b).start()
            gather(ga, rows_a, gsem_a).wait()

            @pl.when(g2 > 0)
            def _():
                writeback(ga - 2, obuf_a, wsem_a).wait()

            compute(rows_a, obuf_a)
            writeback(ga, obuf_a, wsem_a).start()

            @pl.when(g2 < n2 - 1)
            def _():
                gather(ga + 2, rows_a, gsem_a).start()

            gather(ga + 1, rows_b, gsem_b).wait()

            @pl.when(g2 > 0)
            def _():
                writeback(ga - 1, obuf_b, wsem_b).wait()

            compute(rows_b, obuf_b)
            writeback(ga + 1, obuf_b, wsem_b).start()
            return carry

        lax.fori_loop(0, n2, body, 0)
        writeback(2 * n2 - 2, obuf_a, wsem_a).wait()
        writeback(2 * n2 - 1, obuf_b, wsem_b).wait()

    return k


def kernel(x, table, gamma, beta):
    b, s = x.shape
    n = b * s
    out = _make_kernel(n)(x.reshape(n // CHUNK, CHUNK), table, gamma, beta)
    return out.reshape(b, s, D)


# speedup vs baseline: 2.0928x; 1.9286x over previous
import functools

import jax
import jax.numpy as jnp
from jax import lax
from jax.experimental import pallas as pl
from jax.experimental.pallas import tpu as pltpu
from jax.experimental.pallas import tpu_sc as plsc

D = 64
PAD = 64
CHUNK = 128
L = 16
EPS = 1e-5


def _rsqrt(a):
    i = plsc.bitcast(a, jnp.int32)
    i = jnp.int32(0x5F3759DF) - lax.shift_right_logical(i, 1)
    y = plsc.bitcast(i, jnp.float32)
    half = a * 0.5
    for _ in range(3):
        y = y * (1.5 - half * y * y)
    return y


@functools.lru_cache(maxsize=None)
def _make_kernel(n_rows):
    info = plsc.get_sparse_core_info()
    nc, ns = info.num_cores, info.num_subcores
    nw = nc * ns
    rows_per_w = n_rows // nw
    n_chunks = rows_per_w // CHUNK
    n2 = n_chunks // 2
    assert rows_per_w % CHUNK == 0 and n_rows % nw == 0 and n_chunks % 2 == 0
    mesh = plsc.VectorSubcoreMesh(core_axis_name="c", subcore_axis_name="s")

    @functools.partial(
        pl.kernel,
        mesh=mesh,
        out_type=jax.ShapeDtypeStruct((n_rows, D), jnp.float32),
        compiler_params=pltpu.CompilerParams(
            use_tc_tiling_on_sc=False, needs_layout_passes=False
        ),
        scratch_types=[
            pltpu.VMEM((n_chunks, CHUNK), jnp.int32),
            pltpu.VMEM((CHUNK, PAD), jnp.float32),
            pltpu.VMEM((CHUNK, PAD), jnp.float32),
            pltpu.VMEM((CHUNK, PAD), jnp.float32),
            pltpu.VMEM((CHUNK, PAD), jnp.float32),
            pltpu.VMEM((D,), jnp.float32),
            pltpu.VMEM((D,), jnp.float32),
            pltpu.SemaphoreType.DMA,
            pltpu.SemaphoreType.DMA,
            pltpu.SemaphoreType.DMA,
            pltpu.SemaphoreType.DMA,
        ],
    )
    def k(x_hbm, table_hbm, gamma_hbm, beta_hbm, out_hbm,
          idx_v, rows_a, rows_b, obuf_a, obuf_b, gamma_v, beta_v,
          gsem_a, gsem_b, wsem_a, wsem_b):
        wid = lax.axis_index("s") * nc + lax.axis_index("c")
        base0 = wid * rows_per_w
        pltpu.sync_copy(gamma_hbm, gamma_v)
        pltpu.sync_copy(beta_hbm, beta_v)
        pltpu.sync_copy(
            x_hbm.at[pl.ds(wid * n_chunks, n_chunks), :], idx_v
        )
        lanes = lax.iota(jnp.int32, L)

        def gather(g, rows, sem):
            return pltpu.make_async_copy(
                table_hbm.at[idx_v.at[g]], rows.at[:, pl.ds(0, D)], sem
            )

        def compute(rows, obuf):
            gk = [gamma_v[pl.ds(k * L, L)] for k in range(D // L)]
            bk = [beta_v[pl.ds(k * L, L)] for k in range(D // L)]

            def block_body(b, carry2):
                row_idx = b * L + lanes
                acc_s = [jnp.zeros((L,), jnp.float32) for _ in range(4)]
                acc_q = [jnp.zeros((L,), jnp.float32) for _ in range(4)]
                for j in range(D):
                    cd = (lanes + j) & (D - 1)
                    v = plsc.load_gather(rows, [row_idx, cd])
                    acc_s[j % 4] = acc_s[j % 4] + v
                    acc_q[j % 4] = acc_q[j % 4] + v * v
                s = (acc_s[0] + acc_s[1]) + (acc_s[2] + acc_s[3])
                q = (acc_q[0] + acc_q[1]) + (acc_q[2] + acc_q[3])
                mean = s * (1.0 / D)
                var = q * (1.0 / D) - mean * mean
                rstd = _rsqrt(var + EPS)
                for t in range(L):
                    r_s = rstd[t]
                    mr_s = mean[t] * r_s
                    row = b * L + t
                    for kk in range(D // L):
                        v = rows[row, pl.ds(kk * L, L)]
                        o = (v * r_s - mr_s) * gk[kk] + bk[kk]
                        obuf[row, pl.ds(kk * L, L)] = o
                return carry2

            lax.fori_loop(0, CHUNK // L, block_body, 0)

        def writeback(g, obuf, sem):
            return pltpu.make_async_copy(
                obuf.at[:, pl.ds(0, D)],
                out_hbm.at[pl.ds(base0 + g * CHUNK, CHUNK)],
                sem,
            )

        gather(0, rows_a, gsem_a).start()

        def body(g2, carry):
            ga = 2 * g2
            gather(ga + 1, rows_b, gsem_b).start()
            gather(ga, rows_a, gsem_a).wait()

            @pl.when(g2 > 0)
            def _():
                writeback(ga - 2, obuf_a, wsem_a).wait()

            compute(rows_a, obuf_a)
            writeback(ga, obuf_a, wsem_a).start()

            @pl.when(g2 < n2 - 1)
            def _():
                gather(ga + 2, rows_a, gsem_a).start()

            gather(ga + 1, rows_b, gsem_b).wait()

            @pl.when(g2 > 0)
            def _():
                writeback(ga - 1, obuf_b, wsem_b).wait()

            compute(rows_b, obuf_b)
            writeback(ga + 1, obuf_b, wsem_b).start()
            return carry

        lax.fori_loop(0, n2, body, 0)
        writeback(2 * n2 - 2, obuf_a, wsem_a).wait()
        writeback(2 * n2 - 1, obuf_b, wsem_b).wait()

    return k


def kernel(x, table, gamma, beta):
    b, s = x.shape
    n = b * s
    out = _make_kernel(n)(x.reshape(n // CHUNK, CHUNK), table, gamma, beta)
    return out.reshape(b, s, D)
